# baseline (device time: 133772 ns/iter reference)
import jax
import jax.numpy as jnp
from jax import lax
from jax.experimental import pallas as pl
from jax.experimental.pallas import tpu as pltpu

N_DEV = 4
NB = 8

F8 = jnp.float8_e5m2


def kernel(x, w_mat, scale_x, scale_w):
    m_total, k_shard = x.shape
    k_total, n_total = w_mat.shape
    m_per = m_total // N_DEV
    k_per = k_total // N_DEV
    nb = n_total // NB

    def body(x_hbm, w_ref, sx_ref, sw_ref, out_ref,
             xstage_ref, x8_ref, recv_ref, w8_ref,
             copy_sem, send_sems, recv_sems):
        s = pl.program_id(0)
        j = pl.program_id(1)
        my = lax.axis_index("i")

        @pl.when(jnp.logical_and(s == 0, j == 0))
        def _comm_start():
            barrier = pltpu.get_barrier_semaphore()
            for off in range(1, N_DEV):
                pl.semaphore_signal(
                    barrier, inc=1,
                    device_id=((my + off) % N_DEV,),
                    device_id_type=pl.DeviceIdType.MESH,
                )
            pl.semaphore_wait(barrier, N_DEV - 1)

            for b in range(N_DEV):
                cp = pltpu.make_async_copy(
                    x_hbm.at[pl.ds(b * m_per, m_per), :],
                    xstage_ref,
                    copy_sem,
                )
                cp.start()
                cp.wait()
                x8_ref[b] = xstage_ref[...].astype(F8)

            recv_ref[my] = x8_ref[my]

            for off in range(1, N_DEV):
                dst = (my + off) % N_DEV
                rdma = pltpu.make_async_remote_copy(
                    src_ref=x8_ref.at[dst],
                    dst_ref=recv_ref.at[my],
                    send_sem=send_sems.at[off - 1],
                    recv_sem=recv_sems.at[my],
                    device_id=(dst,),
                    device_id_type=pl.DeviceIdType.MESH,
                )
                rdma.start()

        @pl.when(jnp.logical_and(j == 0, s != my))
        def _wait_src():
            pltpu.make_async_remote_copy(
                src_ref=x8_ref.at[0],
                dst_ref=recv_ref.at[s],
                send_sem=send_sems.at[0],
                recv_sem=recv_sems.at[s],
                device_id=(0,),
                device_id_type=pl.DeviceIdType.MESH,
            ).wait_recv()

        w8_ref[...] = w_ref[...].astype(F8)
        part = jnp.dot(recv_ref[s], w8_ref[...],
                       preferred_element_type=jnp.float32)
        ncols = pl.ds(j * nb, nb)

        @pl.when(s == 0)
        def _init():
            out_ref[:, ncols] = part

        @pl.when(jnp.logical_and(s > 0, s < N_DEV - 1))
        def _acc():
            out_ref[:, ncols] += part

        @pl.when(s == N_DEV - 1)
        def _fin():
            out_ref[:, ncols] = (out_ref[:, ncols] + part) * (
                sx_ref[0] * sw_ref[0])

        @pl.when(jnp.logical_and(s == N_DEV - 1, j == NB - 1))
        def _drain():
            for off in range(1, N_DEV):
                pltpu.make_async_remote_copy(
                    src_ref=x8_ref.at[0],
                    dst_ref=recv_ref.at[0],
                    send_sem=send_sems.at[off - 1],
                    recv_sem=recv_sems.at[0],
                    device_id=(0,),
                    device_id_type=pl.DeviceIdType.MESH,
                ).wait_send()

    return pl.pallas_call(
        body,
        grid=(N_DEV, NB),
        in_specs=[
            pl.BlockSpec(memory_space=pltpu.HBM),
            pl.BlockSpec((k_per, nb), lambda s, j: (s, j)),
            pl.BlockSpec(memory_space=pltpu.SMEM),
            pl.BlockSpec(memory_space=pltpu.SMEM),
        ],
        out_specs=pl.BlockSpec((m_per, n_total), lambda s, j: (0, 0)),
        out_shape=jax.ShapeDtypeStruct((m_per, n_total), jnp.float32),
        scratch_shapes=[
            pltpu.VMEM((m_per, k_shard), jnp.float32),
            pltpu.VMEM((N_DEV, m_per, k_shard), F8),
            pltpu.VMEM((N_DEV, m_per, k_shard), F8),
            pltpu.VMEM((k_per, nb), F8),
            pltpu.SemaphoreType.DMA,
            pltpu.SemaphoreType.DMA((N_DEV - 1,)),
            pltpu.SemaphoreType.DMA((N_DEV,)),
        ],
        compiler_params=pltpu.CompilerParams(
            collective_id=0,
            dimension_semantics=("arbitrary", "arbitrary"),
            vmem_limit_bytes=63 * 1024 * 1024,
        ),
    )(x, w_mat, scale_x, scale_w)


# device time: 110228 ns/iter; 1.2136x vs baseline; 1.2136x over previous
import functools

import jax
import jax.numpy as jnp
from jax import lax
from jax.experimental import pallas as pl
from jax.experimental.pallas import tpu as pltpu

N_DEV = 4
NB = 8

F8 = jnp.float8_e5m2


def kernel(x, w_mat, scale_x, scale_w):
    m_total, k_shard = x.shape
    k_total, n_total = w_mat.shape
    m_per = m_total // N_DEV
    k_per = k_total // N_DEV
    nb = n_total // NB

    my_idx = lax.axis_index("i")
    order = (my_idx + jnp.array([0, 1, 3, 2], dtype=jnp.int32)) % N_DEV

    def body(order_ref, x_hbm, w_ref, sx_ref, sw_ref, out_hbm,
             stage_ref, x8_ref, recv_ref, xcur_ref, w8_ref, acc_ref,
             stage_sems, send_sems, recv_sems, out_sems):
        s = pl.program_id(0)
        j = pl.program_id(1)
        my = order_ref[0]
        src = order_ref[s]

        def _out_cp(jj):
            cols = pl.ds(jj * nb, nb)
            return pltpu.make_async_copy(
                acc_ref.at[:, cols], out_hbm.at[:, cols], out_sems.at[jj])

        @pl.when(jnp.logical_and(s == 0, j == 0))
        def _comm_start():
            barrier = pltpu.get_barrier_semaphore()
            for off in range(1, N_DEV):
                pl.semaphore_signal(
                    barrier, inc=1,
                    device_id=((my + off) % N_DEV,),
                    device_id_type=pl.DeviceIdType.MESH,
                )
            pl.semaphore_wait(barrier, N_DEV - 1)

            offs = [1, 0, 3, 2]
            blks = [(my + off) % N_DEV for off in offs]

            def _cp(i):
                return pltpu.make_async_copy(
                    x_hbm.at[pl.ds(blks[i] * m_per, m_per), :],
                    stage_ref.at[i % 2],
                    stage_sems.at[i % 2],
                )
            _cp(0).start()
            for i, off in enumerate(offs):
                if i + 1 < N_DEV:
                    _cp(i + 1).start()
                _cp(i).wait()
                b = blks[i]
                x8_ref[b] = stage_ref[i % 2].astype(F8)
                if off == 0:
                    recv_ref[my] = x8_ref[my]
                else:
                    pltpu.make_async_remote_copy(
                        src_ref=x8_ref.at[b],
                        dst_ref=recv_ref.at[my],
                        send_sem=send_sems.at[off - 1],
                        recv_sem=recv_sems.at[my],
                        device_id=(b,),
                        device_id_type=pl.DeviceIdType.MESH,
                    ).start()

        @pl.when(jnp.logical_and(j == 0, s > 0))
        def _wait_src():
            pltpu.make_async_remote_copy(
                src_ref=x8_ref.at[0],
                dst_ref=recv_ref.at[src],
                send_sem=send_sems.at[0],
                recv_sem=recv_sems.at[src],
                device_id=(0,),
                device_id_type=pl.DeviceIdType.MESH,
            ).wait_recv()

        @pl.when(j == 0)
        def _load_cur():
            xcur_ref[...] = recv_ref[src]

        w8_ref[...] = w_ref[...].astype(F8)
        part = jnp.dot(xcur_ref[...], w8_ref[...],
                       preferred_element_type=jnp.float32)
        ncols = pl.ds(j * nb, nb)

        @pl.when(s == 0)
        def _init():
            acc_ref[:, ncols] = part

        @pl.when(jnp.logical_and(s > 0, s < N_DEV - 1))
        def _acc():
            acc_ref[:, ncols] += part

        @pl.when(s == N_DEV - 1)
        def _fin():
            acc_ref[:, ncols] = (acc_ref[:, ncols] + part) * (
                sx_ref[0] * sw_ref[0])

        for jj in range(NB):
            @pl.when(jnp.logical_and(s == N_DEV - 1, j == jj))
            def _start_out(jj=jj):
                _out_cp(jj).start()

        @pl.when(jnp.logical_and(s == N_DEV - 1, j == NB - 1))
        def _drain():
            for jj in range(NB):
                _out_cp(jj).wait()
            for off in range(1, N_DEV):
                pltpu.make_async_remote_copy(
                    src_ref=x8_ref.at[0],
                    dst_ref=recv_ref.at[0],
                    send_sem=send_sems.at[off - 1],
                    recv_sem=recv_sems.at[0],
                    device_id=(0,),
                    device_id_type=pl.DeviceIdType.MESH,
                ).wait_send()

            @functools.partial(pl.run_scoped,
                               second_barrier=pltpu.SemaphoreType.REGULAR)
            def _(second_barrier):
                for off in range(1, N_DEV):
                    pl.semaphore_signal(
                        second_barrier, inc=1,
                        device_id=((my + off) % N_DEV,),
                        device_id_type=pl.DeviceIdType.MESH,
                    )
                pl.semaphore_wait(second_barrier, N_DEV - 1)

    grid_spec = pltpu.PrefetchScalarGridSpec(
        num_scalar_prefetch=1,
        grid=(N_DEV, NB),
        in_specs=[
            pl.BlockSpec(memory_space=pltpu.HBM),
            pl.BlockSpec((k_per, nb),
                         lambda s, j, order_ref: (order_ref[s], j)),
            pl.BlockSpec(memory_space=pltpu.SMEM),
            pl.BlockSpec(memory_space=pltpu.SMEM),
        ],
        out_specs=pl.BlockSpec(memory_space=pltpu.HBM),
        scratch_shapes=[
            pltpu.VMEM((2, m_per, k_shard), jnp.float32),
            pltpu.VMEM((N_DEV, m_per, k_shard), F8),
            pltpu.VMEM((N_DEV, m_per, k_shard), F8),
            pltpu.VMEM((m_per, k_shard), F8),
            pltpu.VMEM((k_per, nb), F8),
            pltpu.VMEM((m_per, n_total), jnp.float32),
            pltpu.SemaphoreType.DMA((2,)),
            pltpu.SemaphoreType.DMA((N_DEV - 1,)),
            pltpu.SemaphoreType.DMA((N_DEV,)),
            pltpu.SemaphoreType.DMA((NB,)),
        ],
    )
    return pl.pallas_call(
        body,
        grid_spec=grid_spec,
        out_shape=jax.ShapeDtypeStruct((m_per, n_total), jnp.float32),
        compiler_params=pltpu.CompilerParams(
            collective_id=0,
            dimension_semantics=("arbitrary", "arbitrary"),
            vmem_limit_bytes=63 * 1024 * 1024,
        ),
    )(order, x, w_mat, scale_x, scale_w)
